# baseline (speedup 1.0000x reference)
import jax
import jax.numpy as jnp
from jax import lax
from jax.experimental import pallas as pl
from jax.experimental.pallas import tpu as pltpu
from jax.experimental.pallas import tpu_sc as plsc

B = 16384
MAX_FEATURES = 32
VOCAB = 768
FT_OUT = 128
NC = 2
NS = 16
NW = NC * NS
ROWS_PER_W = B // NW


def _sc_body(vals_hbm, stm_hbm, nstm_hbm, tab_hbm, ftb_hbm, ow_hbm, ob_hbm,
             out_hbm,
             tab_v, ftb_v, ow_v, ob_v, vals_v, stm_v, nstm_v, dot_v, out_v):
    wid = lax.axis_index("s") * NC + lax.axis_index("c")
    base = wid * ROWS_PER_W

    pltpu.sync_copy(tab_hbm, tab_v)
    pltpu.sync_copy(ftb_hbm, ftb_v)
    pltpu.sync_copy(ow_hbm, ow_v)
    pltpu.sync_copy(ob_hbm, ob_v)
    off = base * MAX_FEATURES
    pltpu.sync_copy(vals_hbm.at[pl.ds(off, ROWS_PER_W * MAX_FEATURES)], vals_v)
    pltpu.sync_copy(stm_hbm.at[pl.ds(off, ROWS_PER_W * MAX_FEATURES)], stm_v)
    pltpu.sync_copy(nstm_hbm.at[pl.ds(off, ROWS_PER_W * MAX_FEATURES)], nstm_v)
    ob_s = ob_v[pl.ds(0, 16)][0]

    def group_body(g, _):
        def row_body(rr, _):
            r = g * 16 + rr
            s0 = stm_v[pl.ds(r * 32, 16)]
            s1 = stm_v[pl.ds(r * 32 + 16, 16)]
            n0 = nstm_v[pl.ds(r * 32, 16)]
            n1 = nstm_v[pl.ds(r * 32 + 16, 16)]
            v0 = vals_v[pl.ds(r * 32, 16)]
            v1 = vals_v[pl.ds(r * 32 + 16, 16)]

            acc = tuple(ftb_v[pl.ds(k * 16, 32)] for k in range(4)) * 2
            for f in range(MAX_FEATURES):
                b_s = s0[f] if f < 16 else s1[f - 16]
                b_n = n0[f] if f < 16 else n1[f - 16]
                wv = jnp.full((16,), v0[f] if f < 16 else v1[f - 16],
                              jnp.float32)
                w = plsc.pack(wv, wv, format=plsc.PackFormat.INTERLEAVED)
                acc = (
                    tuple(acc[k] + w * tab_v[pl.ds(b_s + k * 16, 32)]
                          for k in range(4))
                    + tuple(acc[4 + k] + w * tab_v[pl.ds(b_n + k * 16, 32)]
                            for k in range(4)))

            dot = jnp.zeros((16,), jnp.float32)
            for k in range(8):
                u = plsc.bitcast(acc[k], jnp.uint32)
                ha = plsc.bitcast(u << 16, jnp.float32)
                hb = plsc.bitcast(u & jnp.uint32(0xFFFF0000), jnp.float32)
                ha = jnp.clip(ha, 0.0, 1.0)
                hb = jnp.clip(hb, 0.0, 1.0)
                dot = dot + ha * ow_v[pl.ds(k * 32, 16)]
                dot = dot + hb * ow_v[pl.ds(k * 32 + 16, 16)]
            dot_v[pl.ds(rr * 16, 16)] = dot
            return 0

        lax.fori_loop(0, 16, row_body, 0)
        col = lax.iota(jnp.int32, 16) * 16
        x = jnp.full((16,), ob_s, jnp.float32)
        for k in range(16):
            x = x + plsc.load_gather(dot_v, [col + k])
        res = 1.0 / (1.0 + jnp.exp(-x))
        out_v[pl.ds(g * 16, 16)] = res
        return 0

    lax.fori_loop(0, ROWS_PER_W // 16, group_body, 0)
    pltpu.sync_copy(out_v, out_hbm.at[pl.ds(base, ROWS_PER_W)])


@jax.jit
def _run(values, stm, nstm, table, ftb, ow, ob):
    mesh = plsc.VectorSubcoreMesh(core_axis_name="c", subcore_axis_name="s")
    f = pl.kernel(
        _sc_body,
        out_type=jax.ShapeDtypeStruct((B,), jnp.float32),
        mesh=mesh,
        scratch_types=[
            pltpu.VMEM((VOCAB * FT_OUT,), jnp.bfloat16),
            pltpu.VMEM((2 * FT_OUT,), jnp.bfloat16),
            pltpu.VMEM((2 * FT_OUT,), jnp.float32),
            pltpu.VMEM((16,), jnp.float32),
            pltpu.VMEM((ROWS_PER_W * MAX_FEATURES,), jnp.float32),
            pltpu.VMEM((ROWS_PER_W * MAX_FEATURES,), jnp.int32),
            pltpu.VMEM((ROWS_PER_W * MAX_FEATURES,), jnp.int32),
            pltpu.VMEM((256,), jnp.float32),
            pltpu.VMEM((ROWS_PER_W,), jnp.float32),
        ],
        compiler_params=pltpu.CompilerParams(needs_layout_passes=False),
    )
    return f(values, stm, nstm, table, ftb, ow, ob)


def kernel(values, stm_indices, nstm_indices, buckets, ft_weight, ft_bias,
           out_weight, out_bias):
    del buckets
    ow = out_weight.reshape(2 * FT_OUT)
    ob = jnp.broadcast_to(out_bias.reshape(1), (16,))
    wt = ft_weight.astype(jnp.bfloat16).reshape(VOCAB // 2, 2, 4, 2, 16)
    tab = wt.transpose(0, 3, 1, 2, 4).reshape(-1)
    fb = ft_bias.astype(jnp.bfloat16).reshape(4, 2, 16)
    z64 = jnp.zeros((4, 16), jnp.bfloat16)
    ftb = jnp.concatenate(
        [fb[:, 0, :], z64, fb[:, 1, :], z64], axis=0).reshape(-1)
    si = stm_indices.astype(jnp.int32)
    ni = nstm_indices.astype(jnp.int32)
    sb = (si >> 1) * 256 + (si & 1) * 64
    nb = (ni >> 1) * 256 + (ni & 1) * 64
    out = _run(values.reshape(-1), sb.reshape(-1), nb.reshape(-1),
               tab, ftb, ow, ob)
    return out.reshape(B, 1)


# speedup vs baseline: 1.4720x; 1.4720x over previous
import jax
import jax.numpy as jnp
from jax import lax
from jax.experimental import pallas as pl
from jax.experimental.pallas import tpu as pltpu
from jax.experimental.pallas import tpu_sc as plsc

B = 16384
MAX_FEATURES = 32
VOCAB = 768
FT_OUT = 128
NC = 2
NS = 16
NW = NC * NS
ROWS_PER_W = B // NW


def _sc_body(vals_hbm, stm_hbm, nstm_hbm, tab_hbm, ftb_hbm, ow_hbm, ob_hbm,
             out_hbm,
             tab_v, ftb_v, ow_v, ob_v, vals_v, stm_v, nstm_v, dot_v, out_v):
    wid = lax.axis_index("s") * NC + lax.axis_index("c")
    base = wid * ROWS_PER_W

    pltpu.sync_copy(tab_hbm, tab_v)
    pltpu.sync_copy(ftb_hbm, ftb_v)
    pltpu.sync_copy(ow_hbm, ow_v)
    pltpu.sync_copy(ob_hbm, ob_v)
    off = base * MAX_FEATURES
    pltpu.sync_copy(vals_hbm.at[pl.ds(off, ROWS_PER_W * MAX_FEATURES)], vals_v)
    pltpu.sync_copy(stm_hbm.at[pl.ds(off, ROWS_PER_W * MAX_FEATURES)], stm_v)
    pltpu.sync_copy(nstm_hbm.at[pl.ds(off, ROWS_PER_W * MAX_FEATURES)], nstm_v)
    ob_s = ob_v[pl.ds(0, 16)][0]

    def group_body(g, _):
        def row_body(rr, _):
            r = g * 16 + rr
            s0 = stm_v[pl.ds(r * 32, 16)]
            s1 = stm_v[pl.ds(r * 32 + 16, 16)]
            n0 = nstm_v[pl.ds(r * 32, 16)]
            n1 = nstm_v[pl.ds(r * 32 + 16, 16)]
            v0 = vals_v[pl.ds(r * 32, 16)]
            v1 = vals_v[pl.ds(r * 32 + 16, 16)]

            acc = tuple(ftb_v[pl.ds(k * 16, 32)] for k in range(4)) * 2
            for f in range(MAX_FEATURES):
                i_s = s0[f] if f < 16 else s1[f - 16]
                i_n = n0[f] if f < 16 else n1[f - 16]
                b_s = (i_s >> 1) * 256 + (i_s & 1) * 64
                b_n = (i_n >> 1) * 256 + (i_n & 1) * 64
                wv = jnp.full((16,), v0[f] if f < 16 else v1[f - 16],
                              jnp.float32)
                w = plsc.pack(wv, wv, format=plsc.PackFormat.INTERLEAVED)
                acc = (
                    tuple(acc[k] + w * tab_v[pl.ds(b_s + k * 16, 32)]
                          for k in range(4))
                    + tuple(acc[4 + k] + w * tab_v[pl.ds(b_n + k * 16, 32)]
                            for k in range(4)))

            dot = jnp.zeros((16,), jnp.float32)
            for k in range(8):
                u = plsc.bitcast(acc[k], jnp.uint32)
                ha = plsc.bitcast(u << 16, jnp.float32)
                hb = plsc.bitcast(u & jnp.uint32(0xFFFF0000), jnp.float32)
                ha = jnp.clip(ha, 0.0, 1.0)
                hb = jnp.clip(hb, 0.0, 1.0)
                dot = dot + ha * ow_v[pl.ds(k * 32, 16)]
                dot = dot + hb * ow_v[pl.ds(k * 32 + 16, 16)]
            dot_v[pl.ds(rr * 16, 16)] = dot
            return 0

        lax.fori_loop(0, 16, row_body, 0)
        col = lax.iota(jnp.int32, 16) * 16
        x = jnp.full((16,), ob_s, jnp.float32)
        for k in range(16):
            x = x + plsc.load_gather(dot_v, [col + k])
        res = 1.0 / (1.0 + jnp.exp(-x))
        out_v[pl.ds(g * 16, 16)] = res
        return 0

    lax.fori_loop(0, ROWS_PER_W // 16, group_body, 0)
    pltpu.sync_copy(out_v, out_hbm.at[pl.ds(base, ROWS_PER_W)])


@jax.jit
def _run(values, stm, nstm, table, ftb, ow, ob):
    mesh = plsc.VectorSubcoreMesh(core_axis_name="c", subcore_axis_name="s")
    f = pl.kernel(
        _sc_body,
        out_type=jax.ShapeDtypeStruct((B,), jnp.float32),
        mesh=mesh,
        scratch_types=[
            pltpu.VMEM((VOCAB * FT_OUT,), jnp.bfloat16),
            pltpu.VMEM((2 * FT_OUT,), jnp.bfloat16),
            pltpu.VMEM((2 * FT_OUT,), jnp.float32),
            pltpu.VMEM((16,), jnp.float32),
            pltpu.VMEM((ROWS_PER_W * MAX_FEATURES,), jnp.float32),
            pltpu.VMEM((ROWS_PER_W * MAX_FEATURES,), jnp.int32),
            pltpu.VMEM((ROWS_PER_W * MAX_FEATURES,), jnp.int32),
            pltpu.VMEM((256,), jnp.float32),
            pltpu.VMEM((ROWS_PER_W,), jnp.float32),
        ],
        compiler_params=pltpu.CompilerParams(needs_layout_passes=False),
    )
    return f(values, stm, nstm, table, ftb, ow, ob)


def kernel(values, stm_indices, nstm_indices, buckets, ft_weight, ft_bias,
           out_weight, out_bias):
    del buckets
    ow = out_weight.reshape(2 * FT_OUT)
    ob = jnp.broadcast_to(out_bias.reshape(1), (16,))
    wt = ft_weight.astype(jnp.bfloat16).reshape(VOCAB // 2, 2, 4, 2, 16)
    tab = wt.transpose(0, 3, 1, 2, 4).reshape(-1)
    fb = ft_bias.astype(jnp.bfloat16).reshape(4, 2, 16)
    z64 = jnp.zeros((4, 16), jnp.bfloat16)
    ftb = jnp.concatenate(
        [fb[:, 0, :], z64, fb[:, 1, :], z64], axis=0).reshape(-1)
    out = _run(values.reshape(-1),
               stm_indices.astype(jnp.int32).reshape(-1),
               nstm_indices.astype(jnp.int32).reshape(-1),
               tab, ftb, ow, ob)
    return out.reshape(B, 1)
